# Initial kernel scaffold; baseline (speedup 1.0000x reference)
import functools

import jax
import jax.numpy as jnp
from jax import lax
from jax.experimental import pallas as pl
from jax.experimental.pallas import tpu as pltpu
from jax.experimental.pallas import tpu_sc as plsc

N_ROWS = 100000
N_FEAT = 128
N_SEG = 512
LANES = 16

N_FG = N_FEAT // LANES
N_RG = 4
ROWS_PER_RG = N_ROWS // N_RG
CHUNK = 1000
N_CHUNKS = ROWS_PER_RG // CHUNK

_mesh = plsc.VectorSubcoreMesh(core_axis_name="c", subcore_axis_name="s")

NEG_INF = jnp.float32(-jnp.inf)


@functools.partial(
    pl.kernel,
    mesh=_mesh,
    out_type=[
        jax.ShapeDtypeStruct((N_RG, N_SEG, N_FEAT), jnp.float32),
        jax.ShapeDtypeStruct((N_RG, N_SEG, N_FEAT), jnp.float32),
        jax.ShapeDtypeStruct((N_RG, N_SEG, LANES), jnp.float32),
    ],
    scratch_types=[
        pltpu.VMEM((CHUNK, LANES), jnp.float32),
        pltpu.SMEM((CHUNK,), jnp.int32),
        pltpu.VMEM((N_SEG, LANES), jnp.float32),
        pltpu.VMEM((N_SEG, LANES), jnp.float32),
        pltpu.VMEM((N_SEG, LANES), jnp.float32),
    ],
)
def _phase_a(node_hbm, idx_hbm, sums_hbm, maxs_hbm, cnts_hbm,
             rowbuf, idx_sm, sumtab, maxtab, cnttab):
    c = lax.axis_index("c")
    s = lax.axis_index("s")
    wid = s * 2 + c
    fg = wid % N_FG
    rg = wid // N_FG
    row0 = rg * ROWS_PER_RG
    col0 = fg * LANES

    zeros = jnp.zeros((LANES,), jnp.float32)
    ninf = jnp.full((LANES,), NEG_INF)

    def init_body(i, _):
        sumtab[i, :] = zeros
        maxtab[i, :] = ninf
        cnttab[i, :] = zeros
        return 0

    lax.fori_loop(0, N_SEG, init_body, 0)

    def row_body(i, carry):
        cur, acc_s, acc_m, cnt = carry
        sid = idx_sm[i]
        changed = sid != cur

        @pl.when(jnp.logical_and(changed, cnt > 0.0))
        def _flush():
            sumtab[cur, :] = acc_s
            maxtab[cur, :] = acc_m
            cnttab[cur, :] = jnp.full((LANES,), cnt)

        v = rowbuf[i, :]
        acc_s = jnp.where(changed, v, acc_s + v)
        acc_m = jnp.where(changed, v, jnp.maximum(acc_m, v))
        cnt = jnp.where(changed, 1.0, cnt + 1.0)
        return sid, acc_s, acc_m, cnt

    def chunk_body(ck, carry):
        base = row0 + ck * CHUNK
        pltpu.sync_copy(node_hbm.at[pl.ds(base, CHUNK), pl.ds(col0, LANES)],
                        rowbuf)
        pltpu.sync_copy(idx_hbm.at[pl.ds(base, CHUNK)], idx_sm)
        return lax.fori_loop(0, CHUNK, row_body, carry)

    carry0 = (jnp.int32(0), zeros, ninf, jnp.float32(0.0))
    cur, acc_s, acc_m, cnt = lax.fori_loop(0, N_CHUNKS, chunk_body, carry0)

    sumtab[cur, :] = acc_s
    maxtab[cur, :] = acc_m
    cnttab[cur, :] = jnp.full((LANES,), cnt)

    pltpu.sync_copy(sumtab, sums_hbm.at[rg, :, pl.ds(col0, LANES)])
    pltpu.sync_copy(maxtab, maxs_hbm.at[rg, :, pl.ds(col0, LANES)])

    @pl.when(fg == 0)
    def _store_counts():
        pltpu.sync_copy(cnttab, cnts_hbm.at[rg])


SEG_PER_W = N_SEG // 32


@functools.partial(
    pl.kernel,
    mesh=_mesh,
    out_type=jax.ShapeDtypeStruct((N_SEG, 2 * N_FEAT), jnp.float32),
    scratch_types=[
        pltpu.VMEM((N_RG, SEG_PER_W, N_FEAT), jnp.float32),
        pltpu.VMEM((N_RG, SEG_PER_W, N_FEAT), jnp.float32),
        pltpu.VMEM((N_RG, SEG_PER_W, LANES), jnp.float32),
        pltpu.VMEM((SEG_PER_W, 2 * N_FEAT), jnp.float32),
    ],
)
def _phase_b(sums_hbm, maxs_hbm, cnts_hbm, out_hbm, sbuf, mbuf, cbuf, obuf):
    c = lax.axis_index("c")
    s = lax.axis_index("s")
    wid = s * 2 + c
    seg0 = wid * SEG_PER_W

    for rg in range(N_RG):
        pltpu.sync_copy(sums_hbm.at[rg, pl.ds(seg0, SEG_PER_W), :],
                        sbuf.at[rg])
        pltpu.sync_copy(maxs_hbm.at[rg, pl.ds(seg0, SEG_PER_W), :],
                        mbuf.at[rg])
        pltpu.sync_copy(cnts_hbm.at[rg, pl.ds(seg0, SEG_PER_W), :],
                        cbuf.at[rg])

    def seg_body(k, _):
        cnt = cbuf[0, k, :] + cbuf[1, k, :] + cbuf[2, k, :] + cbuf[3, k, :]
        denom = jnp.maximum(cnt, 1.0)
        for f in range(N_FG):
            fs = pl.ds(f * LANES, LANES)
            ssum = (sbuf[0, k, fs] + sbuf[1, k, fs]
                    + sbuf[2, k, fs] + sbuf[3, k, fs])
            obuf[k, fs] = ssum / denom
            mx = jnp.maximum(jnp.maximum(mbuf[0, k, fs], mbuf[1, k, fs]),
                             jnp.maximum(mbuf[2, k, fs], mbuf[3, k, fs]))
            obuf[k, pl.ds(N_FEAT + f * LANES, LANES)] = mx
        return 0

    lax.fori_loop(0, SEG_PER_W, seg_body, 0)
    pltpu.sync_copy(obuf, out_hbm.at[pl.ds(seg0, SEG_PER_W), :])


def kernel(node_repr, batch_idx):
    batch_idx = batch_idx.astype(jnp.int32)
    sums, maxs, cnts = _phase_a(node_repr, batch_idx)
    return _phase_b(sums, maxs, cnts)


# speedup vs baseline: 1.6443x; 1.6443x over previous
import functools

import jax
import jax.numpy as jnp
from jax import lax
from jax.experimental import pallas as pl
from jax.experimental.pallas import tpu as pltpu
from jax.experimental.pallas import tpu_sc as plsc

N_ROWS = 100000
N_FEAT = 128
N_SEG = 512
LANES = 16

N_FG = N_FEAT // LANES
N_RG = 4
ROWS_PER_RG = N_ROWS // N_RG
CHUNK = 5000
N_CHUNKS = ROWS_PER_RG // CHUNK

_mesh = plsc.VectorSubcoreMesh(core_axis_name="c", subcore_axis_name="s")
_params = pltpu.CompilerParams(use_tc_tiling_on_sc=False)

NEG_INF = float("-inf")


@functools.partial(
    pl.kernel,
    mesh=_mesh,
    compiler_params=_params,
    out_type=[
        jax.ShapeDtypeStruct((N_RG, N_SEG, N_FEAT), jnp.float32),
        jax.ShapeDtypeStruct((N_RG, N_SEG, N_FEAT), jnp.float32),
        jax.ShapeDtypeStruct((N_RG, N_SEG, LANES), jnp.float32),
    ],
    scratch_types=[
        pltpu.VMEM((CHUNK, LANES), jnp.float32),
        pltpu.VMEM((CHUNK + LANES,), jnp.int32),
        pltpu.VMEM((N_SEG, LANES), jnp.float32),
        pltpu.VMEM((N_SEG, LANES), jnp.float32),
        pltpu.VMEM((N_SEG, LANES), jnp.float32),
    ],
)
def _phase_a(node_hbm, idx_hbm, sums_hbm, maxs_hbm, cnts_hbm,
             rowbuf, idxv, sumtab, maxtab, cnttab):
    c = lax.axis_index("c")
    s = lax.axis_index("s")
    wid = s * 2 + c
    fg = wid % N_FG
    rg = wid // N_FG
    row0 = rg * ROWS_PER_RG
    col0 = fg * LANES

    zeros = jnp.zeros((LANES,), jnp.float32)
    ninf = jnp.full((LANES,), NEG_INF)

    def init_body(i, _):
        sumtab[i] = zeros
        maxtab[i] = ninf
        cnttab[i] = zeros
        return 0

    lax.fori_loop(0, N_SEG, init_body, 0)

    def row_body(i, carry):
        cur, acc_s, acc_m, cnt = carry
        sid = idxv[pl.ds(i, LANES)][0]
        changed = sid != cur
        v = rowbuf[i]
        acc_s = jnp.where(changed, v, acc_s + v)
        acc_m = jnp.where(changed, v, jnp.maximum(acc_m, v))
        cnt = jnp.where(changed, 1.0, cnt + 1.0)
        sumtab[sid] = acc_s
        maxtab[sid] = acc_m
        cnttab[sid] = jnp.full((LANES,), cnt)
        return sid, acc_s, acc_m, cnt

    def chunk_body(ck, carry):
        base = row0 + ck * CHUNK
        pltpu.sync_copy(node_hbm.at[pl.ds(base, CHUNK), pl.ds(col0, LANES)],
                        rowbuf)
        pltpu.sync_copy(idx_hbm.at[pl.ds(base, CHUNK)],
                        idxv.at[pl.ds(0, CHUNK)])
        return lax.fori_loop(0, CHUNK, row_body, carry)

    carry0 = (jnp.int32(0), zeros, ninf, jnp.float32(0.0))
    lax.fori_loop(0, N_CHUNKS, chunk_body, carry0)

    pltpu.sync_copy(sumtab, sums_hbm.at[rg, :, pl.ds(col0, LANES)])
    pltpu.sync_copy(maxtab, maxs_hbm.at[rg, :, pl.ds(col0, LANES)])

    @pl.when(fg == 0)
    def _store_counts():
        pltpu.sync_copy(cnttab, cnts_hbm.at[rg])


N_WORKERS = 32
SEG_PER_W = N_SEG // N_WORKERS


@functools.partial(
    pl.kernel,
    mesh=_mesh,
    compiler_params=_params,
    out_type=jax.ShapeDtypeStruct((N_SEG * 2 * N_FG, LANES), jnp.float32),
    scratch_types=[
        pltpu.VMEM((N_RG * SEG_PER_W * N_FG, LANES), jnp.float32),
        pltpu.VMEM((N_RG * SEG_PER_W * N_FG, LANES), jnp.float32),
        pltpu.VMEM((N_RG * SEG_PER_W, LANES), jnp.float32),
        pltpu.VMEM((SEG_PER_W * 2 * N_FG, LANES), jnp.float32),
    ],
)
def _phase_b(sums_hbm, maxs_hbm, cnts_hbm, out_hbm, sbuf, mbuf, cbuf, obuf):
    c = lax.axis_index("c")
    s = lax.axis_index("s")
    wid = s * 2 + c
    seg0 = wid * SEG_PER_W
    W = SEG_PER_W * N_FG

    for rg in range(N_RG):
        pltpu.sync_copy(
            sums_hbm.at[pl.ds((rg * N_SEG + seg0) * N_FG, W)],
            sbuf.at[pl.ds(rg * W, W)])
        pltpu.sync_copy(
            maxs_hbm.at[pl.ds((rg * N_SEG + seg0) * N_FG, W)],
            mbuf.at[pl.ds(rg * W, W)])
        pltpu.sync_copy(
            cnts_hbm.at[pl.ds(rg * N_SEG + seg0, SEG_PER_W)],
            cbuf.at[pl.ds(rg * SEG_PER_W, SEG_PER_W)])

    def seg_body(k, _):
        cnt = (cbuf[k] + cbuf[SEG_PER_W + k]
               + cbuf[2 * SEG_PER_W + k] + cbuf[3 * SEG_PER_W + k])
        denom = jnp.maximum(cnt, 1.0)
        for f in range(N_FG):
            r = k * N_FG + f
            ssum = sbuf[r] + sbuf[W + r] + sbuf[2 * W + r] + sbuf[3 * W + r]
            obuf[k * 2 * N_FG + f] = ssum / denom
            mx = jnp.maximum(jnp.maximum(mbuf[r], mbuf[W + r]),
                             jnp.maximum(mbuf[2 * W + r], mbuf[3 * W + r]))
            obuf[k * 2 * N_FG + N_FG + f] = mx
        return 0

    lax.fori_loop(0, SEG_PER_W, seg_body, 0)
    pltpu.sync_copy(obuf, out_hbm.at[pl.ds(seg0 * 2 * N_FG,
                                           SEG_PER_W * 2 * N_FG)])


def kernel(node_repr, batch_idx):
    batch_idx = batch_idx.astype(jnp.int32)
    sums, maxs, cnts = _phase_a(node_repr, batch_idx)
    out = _phase_b(sums.reshape(N_RG * N_SEG * N_FG, LANES),
                   maxs.reshape(N_RG * N_SEG * N_FG, LANES),
                   cnts.reshape(N_RG * N_SEG, LANES))
    return out.reshape(N_SEG, 2 * N_FEAT)


# speedup vs baseline: 4.4445x; 2.7029x over previous
import functools

import jax
import jax.numpy as jnp
from jax import lax
from jax.experimental import pallas as pl
from jax.experimental.pallas import tpu as pltpu
from jax.experimental.pallas import tpu_sc as plsc

N_ROWS = 100000
N_FEAT = 128
N_SEG = 512
LANES = 16

N_FG = N_FEAT // LANES
N_RG = 4
ROWS_PER_RG = N_ROWS // N_RG
CHUNK = 5000
N_CHUNKS = ROWS_PER_RG // CHUNK

_mesh = plsc.VectorSubcoreMesh(core_axis_name="c", subcore_axis_name="s")
_params = pltpu.CompilerParams(use_tc_tiling_on_sc=False)

NEG_INF = float("-inf")


@functools.partial(
    pl.kernel,
    mesh=_mesh,
    compiler_params=_params,
    out_type=[
        jax.ShapeDtypeStruct((N_RG, N_SEG, N_FEAT), jnp.float32),
        jax.ShapeDtypeStruct((N_RG, N_SEG, N_FEAT), jnp.float32),
        jax.ShapeDtypeStruct((N_RG, N_SEG, LANES), jnp.float32),
    ],
    scratch_types=[
        pltpu.VMEM((CHUNK, LANES), jnp.float32),
        pltpu.VMEM((CHUNK + LANES,), jnp.int32),
        pltpu.VMEM((N_SEG, LANES), jnp.float32),
        pltpu.VMEM((N_SEG, LANES), jnp.float32),
        pltpu.VMEM((N_SEG, LANES), jnp.float32),
        pltpu.VMEM((2, LANES), jnp.float32),
    ],
)
def _phase_a(node_hbm, idx_hbm, sums_hbm, maxs_hbm, cnts_hbm,
             rowbuf, idxv, sumtab, maxtab, cnttab, accb):
    c = lax.axis_index("c")
    s = lax.axis_index("s")
    wid = s * 2 + c
    fg = wid % N_FG
    rg = wid // N_FG
    row0 = rg * ROWS_PER_RG
    col0 = fg * LANES

    zeros = jnp.zeros((LANES,), jnp.float32)
    ninf = jnp.full((LANES,), NEG_INF)

    def init_body(i, _):
        sumtab[i] = zeros
        maxtab[i] = ninf
        cnttab[i] = zeros
        return 0

    lax.fori_loop(0, N_SEG, init_body, 0)

    def row_body(i, carry):
        cur, cnt = carry
        sid = idxv[pl.ds(i, LANES)][0]
        changed = sid != cur
        v = rowbuf[i]
        acc_s = jnp.where(changed, v, accb[0] + v)
        acc_m = jnp.where(changed, v, jnp.maximum(accb[1], v))
        cnt = jnp.where(changed, 1.0, cnt + 1.0)
        accb[0] = acc_s
        accb[1] = acc_m
        sumtab[sid] = acc_s
        maxtab[sid] = acc_m
        cnttab[sid] = jnp.full((LANES,), cnt)
        return sid, cnt

    BLK = 16
    N_BLKS = CHUNK // BLK
    TAIL = CHUNK - N_BLKS * BLK

    def blk_body(b, carry):
        bbase = b * BLK
        ids = idxv[pl.ds(bbase, LANES)]
        first = ids[0]
        last = ids[LANES - 1]

        def fast(carry):
            cur, cnt = carry
            vs = [rowbuf[bbase + j] for j in range(BLK)]
            ss = vs
            mm = vs
            while len(ss) > 1:
                ss = [ss[2 * j] + ss[2 * j + 1] for j in range(len(ss) // 2)]
                mm = [jnp.maximum(mm[2 * j], mm[2 * j + 1])
                      for j in range(len(mm) // 2)]
            bsum, bmax = ss[0], mm[0]
            changed = first != cur
            acc_s = jnp.where(changed, bsum, accb[0] + bsum)
            acc_m = jnp.where(changed, bmax, jnp.maximum(accb[1], bmax))
            cnt = jnp.where(changed, float(BLK), cnt + float(BLK))
            accb[0] = acc_s
            accb[1] = acc_m
            sumtab[first] = acc_s
            maxtab[first] = acc_m
            cnttab[first] = jnp.full((LANES,), cnt)
            return first, cnt

        def slow(carry):
            return lax.fori_loop(bbase, bbase + BLK, row_body, carry)

        return lax.cond(first == last, fast, slow, carry)

    def chunk_body(ck, carry):
        base = row0 + ck * CHUNK
        pltpu.sync_copy(node_hbm.at[pl.ds(base, CHUNK), pl.ds(col0, LANES)],
                        rowbuf)
        pltpu.sync_copy(idx_hbm.at[pl.ds(base, CHUNK)],
                        idxv.at[pl.ds(0, CHUNK)])
        carry = lax.fori_loop(0, N_BLKS, blk_body, carry)
        if TAIL:
            carry = lax.fori_loop(N_BLKS * BLK, CHUNK, row_body, carry)
        return carry

    accb[0] = zeros
    accb[1] = ninf
    carry0 = (jnp.int32(0), jnp.float32(0.0))
    lax.fori_loop(0, N_CHUNKS, chunk_body, carry0)

    pltpu.sync_copy(sumtab, sums_hbm.at[rg, :, pl.ds(col0, LANES)])
    pltpu.sync_copy(maxtab, maxs_hbm.at[rg, :, pl.ds(col0, LANES)])

    @pl.when(fg == 0)
    def _store_counts():
        pltpu.sync_copy(cnttab, cnts_hbm.at[rg])


N_WORKERS = 32
SEG_PER_W = N_SEG // N_WORKERS


@functools.partial(
    pl.kernel,
    mesh=_mesh,
    compiler_params=_params,
    out_type=jax.ShapeDtypeStruct((N_SEG * 2 * N_FG, LANES), jnp.float32),
    scratch_types=[
        pltpu.VMEM((N_RG * SEG_PER_W * N_FG, LANES), jnp.float32),
        pltpu.VMEM((N_RG * SEG_PER_W * N_FG, LANES), jnp.float32),
        pltpu.VMEM((N_RG * SEG_PER_W, LANES), jnp.float32),
        pltpu.VMEM((SEG_PER_W * 2 * N_FG, LANES), jnp.float32),
    ],
)
def _phase_b(sums_hbm, maxs_hbm, cnts_hbm, out_hbm, sbuf, mbuf, cbuf, obuf):
    c = lax.axis_index("c")
    s = lax.axis_index("s")
    wid = s * 2 + c
    seg0 = wid * SEG_PER_W
    W = SEG_PER_W * N_FG

    for rg in range(N_RG):
        pltpu.sync_copy(
            sums_hbm.at[pl.ds((rg * N_SEG + seg0) * N_FG, W)],
            sbuf.at[pl.ds(rg * W, W)])
        pltpu.sync_copy(
            maxs_hbm.at[pl.ds((rg * N_SEG + seg0) * N_FG, W)],
            mbuf.at[pl.ds(rg * W, W)])
        pltpu.sync_copy(
            cnts_hbm.at[pl.ds(rg * N_SEG + seg0, SEG_PER_W)],
            cbuf.at[pl.ds(rg * SEG_PER_W, SEG_PER_W)])

    def seg_body(k, _):
        cnt = (cbuf[k] + cbuf[SEG_PER_W + k]
               + cbuf[2 * SEG_PER_W + k] + cbuf[3 * SEG_PER_W + k])
        denom = jnp.maximum(cnt, 1.0)
        for f in range(N_FG):
            r = k * N_FG + f
            ssum = sbuf[r] + sbuf[W + r] + sbuf[2 * W + r] + sbuf[3 * W + r]
            obuf[k * 2 * N_FG + f] = ssum / denom
            mx = jnp.maximum(jnp.maximum(mbuf[r], mbuf[W + r]),
                             jnp.maximum(mbuf[2 * W + r], mbuf[3 * W + r]))
            obuf[k * 2 * N_FG + N_FG + f] = mx
        return 0

    lax.fori_loop(0, SEG_PER_W, seg_body, 0)
    pltpu.sync_copy(obuf, out_hbm.at[pl.ds(seg0 * 2 * N_FG,
                                           SEG_PER_W * 2 * N_FG)])


def kernel(node_repr, batch_idx):
    batch_idx = batch_idx.astype(jnp.int32)
    sums, maxs, cnts = _phase_a(node_repr, batch_idx)
    out = _phase_b(sums.reshape(N_RG * N_SEG * N_FG, LANES),
                   maxs.reshape(N_RG * N_SEG * N_FG, LANES),
                   cnts.reshape(N_RG * N_SEG, LANES))
    return out.reshape(N_SEG, 2 * N_FEAT)


# speedup vs baseline: 5.4772x; 1.2323x over previous
import functools

import jax
import jax.numpy as jnp
from jax import lax
from jax.experimental import pallas as pl
from jax.experimental.pallas import tpu as pltpu
from jax.experimental.pallas import tpu_sc as plsc

N_ROWS = 100000
N_FEAT = 128
N_SEG = 512
LANES = 16

N_FG = N_FEAT // LANES
N_RG = 4
ROWS_PER_RG = N_ROWS // N_RG
CHUNK = 1000
N_CHUNKS = ROWS_PER_RG // CHUNK

_mesh = plsc.VectorSubcoreMesh(core_axis_name="c", subcore_axis_name="s")
_params = pltpu.CompilerParams(use_tc_tiling_on_sc=False)

NEG_INF = float("-inf")


@functools.partial(
    pl.kernel,
    mesh=_mesh,
    compiler_params=_params,
    out_type=[
        jax.ShapeDtypeStruct((N_RG, N_SEG, N_FEAT), jnp.float32),
        jax.ShapeDtypeStruct((N_RG, N_SEG, N_FEAT), jnp.float32),
        jax.ShapeDtypeStruct((N_RG, N_SEG, LANES), jnp.float32),
    ],
    scratch_types=[
        pltpu.VMEM((2 * CHUNK, LANES), jnp.float32),
        pltpu.VMEM((2 * CHUNK + LANES,), jnp.int32),
        pltpu.VMEM((N_SEG, LANES), jnp.float32),
        pltpu.VMEM((N_SEG, LANES), jnp.float32),
        pltpu.VMEM((N_SEG, LANES), jnp.float32),
        pltpu.VMEM((2, LANES), jnp.float32),
        pltpu.SemaphoreType.DMA,
        pltpu.SemaphoreType.DMA,
        pltpu.SemaphoreType.DMA,
        pltpu.SemaphoreType.DMA,
    ],
)
def _phase_a(node_hbm, idx_hbm, sums_hbm, maxs_hbm, cnts_hbm,
             rowbuf, idxv, sumtab, maxtab, cnttab, accb,
             semr0, semr1, semi0, semi1):
    c = lax.axis_index("c")
    s = lax.axis_index("s")
    wid = s * 2 + c
    fg = wid % N_FG
    rg = wid // N_FG
    row0 = rg * ROWS_PER_RG
    col0 = fg * LANES

    zeros = jnp.zeros((LANES,), jnp.float32)
    ninf = jnp.full((LANES,), NEG_INF)

    def init_body(i, _):
        sumtab[i] = zeros
        maxtab[i] = ninf
        cnttab[i] = zeros
        return 0

    lax.fori_loop(0, N_SEG, init_body, 0)

    def row_body(i, carry):
        cur, cnt = carry
        sid = idxv[pl.ds(i, LANES)][0]
        changed = sid != cur
        v = rowbuf[i]
        acc_s = jnp.where(changed, v, accb[0] + v)
        acc_m = jnp.where(changed, v, jnp.maximum(accb[1], v))
        cnt = jnp.where(changed, 1.0, cnt + 1.0)
        accb[0] = acc_s
        accb[1] = acc_m
        sumtab[sid] = acc_s
        maxtab[sid] = acc_m
        cnttab[sid] = jnp.full((LANES,), cnt)
        return sid, cnt

    BLK = 16
    N_BLKS = CHUNK // BLK
    TAIL = CHUNK - N_BLKS * BLK

    _sems = ((semr0, semi0), (semr1, semi1))

    def _row_cp(ck, half):
        base = row0 + ck * CHUNK
        return pltpu.make_async_copy(
            node_hbm.at[pl.ds(base, CHUNK), pl.ds(col0, LANES)],
            rowbuf.at[pl.ds(half * CHUNK, CHUNK)], _sems[half][0])

    def _idx_cp(ck, half):
        base = row0 + ck * CHUNK
        return pltpu.make_async_copy(
            idx_hbm.at[pl.ds(base, CHUNK)],
            idxv.at[pl.ds(half * CHUNK, CHUNK)], _sems[half][1])

    def blk_body_for(off):
      def blk_body(b, carry):
        ids_n = idxv[pl.ds(off + (b + 1) * BLK, LANES)]
        nfirst = ids_n[0]
        nlast = ids_n[LANES - 1]
        bbase = off + b * BLK
        cur, cnt, first, last = carry

        def fast(carry):
            cur, cnt = carry
            changed = first != cur
            vs = [rowbuf[bbase + j] for j in range(BLK)]
            ss = vs
            mm = vs
            while len(ss) > 1:
                ss = [ss[2 * j] + ss[2 * j + 1] for j in range(len(ss) // 2)]
                mm = [jnp.maximum(mm[2 * j], mm[2 * j + 1])
                      for j in range(len(mm) // 2)]
            bsum, bmax = ss[0], mm[0]
            acc_s = jnp.where(changed, bsum, accb[0] + bsum)
            acc_m = jnp.where(changed, bmax, jnp.maximum(accb[1], bmax))
            cnt = jnp.where(changed, float(BLK), cnt + float(BLK))
            accb[0] = acc_s
            accb[1] = acc_m
            sumtab[first] = acc_s
            maxtab[first] = acc_m
            cnttab[first] = jnp.full((LANES,), cnt)
            return first, cnt

        def slow(carry):
            return lax.fori_loop(bbase, bbase + BLK, row_body, carry)

        cur, cnt = lax.cond(first == last, fast, slow, (cur, cnt))
        return cur, cnt, nfirst, nlast
      return blk_body

    def chunk_body(ck, carry):
        buf = jnp.bitwise_and(ck, 1)

        @pl.when(buf == 0)
        def _wait0():
            _row_cp(ck, 0).wait()
            _idx_cp(ck, 0).wait()

        @pl.when(buf == 1)
        def _wait1():
            _row_cp(ck, 1).wait()
            _idx_cp(ck, 1).wait()

        nxt = ck + 1

        @pl.when(jnp.logical_and(buf == 0, nxt < N_CHUNKS))
        def _issue1():
            _row_cp(nxt, 1).start()
            _idx_cp(nxt, 1).start()

        @pl.when(jnp.logical_and(buf == 1, nxt < N_CHUNKS))
        def _issue0():
            _row_cp(nxt, 0).start()
            _idx_cp(nxt, 0).start()

        off = buf * CHUNK
        ids0 = idxv[pl.ds(off, LANES)]
        cur, cnt = carry
        carry4 = (cur, cnt, ids0[0], ids0[LANES - 1])
        cur, cnt, _, _ = lax.fori_loop(0, N_BLKS, blk_body_for(off), carry4)
        if TAIL:
            cur, cnt = lax.fori_loop(off + N_BLKS * BLK, off + CHUNK,
                                     row_body, (cur, cnt))
        return cur, cnt

    accb[0] = zeros
    accb[1] = ninf
    _row_cp(0, 0).start()
    _idx_cp(0, 0).start()
    carry0 = (jnp.int32(0), jnp.float32(0.0))
    lax.fori_loop(0, N_CHUNKS, chunk_body, carry0)

    pltpu.sync_copy(sumtab, sums_hbm.at[rg, :, pl.ds(col0, LANES)])
    pltpu.sync_copy(maxtab, maxs_hbm.at[rg, :, pl.ds(col0, LANES)])

    @pl.when(fg == 0)
    def _store_counts():
        pltpu.sync_copy(cnttab, cnts_hbm.at[rg])


N_WORKERS = 32
SEG_PER_W = N_SEG // N_WORKERS


@functools.partial(
    pl.kernel,
    mesh=_mesh,
    compiler_params=_params,
    out_type=jax.ShapeDtypeStruct((N_SEG * 2 * N_FG, LANES), jnp.float32),
    scratch_types=[
        pltpu.VMEM((N_RG * SEG_PER_W * N_FG, LANES), jnp.float32),
        pltpu.VMEM((N_RG * SEG_PER_W * N_FG, LANES), jnp.float32),
        pltpu.VMEM((N_RG * SEG_PER_W, LANES), jnp.float32),
        pltpu.VMEM((SEG_PER_W * 2 * N_FG, LANES), jnp.float32),
    ],
)
def _phase_b(sums_hbm, maxs_hbm, cnts_hbm, out_hbm, sbuf, mbuf, cbuf, obuf):
    c = lax.axis_index("c")
    s = lax.axis_index("s")
    wid = s * 2 + c
    seg0 = wid * SEG_PER_W
    W = SEG_PER_W * N_FG

    for rg in range(N_RG):
        pltpu.sync_copy(
            sums_hbm.at[pl.ds((rg * N_SEG + seg0) * N_FG, W)],
            sbuf.at[pl.ds(rg * W, W)])
        pltpu.sync_copy(
            maxs_hbm.at[pl.ds((rg * N_SEG + seg0) * N_FG, W)],
            mbuf.at[pl.ds(rg * W, W)])
        pltpu.sync_copy(
            cnts_hbm.at[pl.ds(rg * N_SEG + seg0, SEG_PER_W)],
            cbuf.at[pl.ds(rg * SEG_PER_W, SEG_PER_W)])

    def seg_body(k, _):
        cnt = (cbuf[k] + cbuf[SEG_PER_W + k]
               + cbuf[2 * SEG_PER_W + k] + cbuf[3 * SEG_PER_W + k])
        denom = jnp.maximum(cnt, 1.0)
        for f in range(N_FG):
            r = k * N_FG + f
            ssum = sbuf[r] + sbuf[W + r] + sbuf[2 * W + r] + sbuf[3 * W + r]
            obuf[k * 2 * N_FG + f] = ssum / denom
            mx = jnp.maximum(jnp.maximum(mbuf[r], mbuf[W + r]),
                             jnp.maximum(mbuf[2 * W + r], mbuf[3 * W + r]))
            obuf[k * 2 * N_FG + N_FG + f] = mx
        return 0

    lax.fori_loop(0, SEG_PER_W, seg_body, 0)
    pltpu.sync_copy(obuf, out_hbm.at[pl.ds(seg0 * 2 * N_FG,
                                           SEG_PER_W * 2 * N_FG)])


def kernel(node_repr, batch_idx):
    batch_idx = batch_idx.astype(jnp.int32)
    sums, maxs, cnts = _phase_a(node_repr, batch_idx)
    out = _phase_b(sums.reshape(N_RG * N_SEG * N_FG, LANES),
                   maxs.reshape(N_RG * N_SEG * N_FG, LANES),
                   cnts.reshape(N_RG * N_SEG, LANES))
    return out.reshape(N_SEG, 2 * N_FEAT)


# speedup vs baseline: 5.5471x; 1.0128x over previous
import functools

import jax
import jax.numpy as jnp
from jax import lax
from jax.experimental import pallas as pl
from jax.experimental.pallas import tpu as pltpu
from jax.experimental.pallas import tpu_sc as plsc

N_ROWS = 100000
N_FEAT = 128
N_SEG = 512
LANES = 16

N_FG = N_FEAT // LANES
N_RG = 4
ROWS_PER_RG = N_ROWS // N_RG
CHUNK = 2496
N_FULL = ROWS_PER_RG // CHUNK
TAIL_ROWS = ROWS_PER_RG - N_FULL * CHUNK

_mesh = plsc.VectorSubcoreMesh(core_axis_name="c", subcore_axis_name="s")
_params = pltpu.CompilerParams(use_tc_tiling_on_sc=False)

NEG_INF = float("-inf")


@functools.partial(
    pl.kernel,
    mesh=_mesh,
    compiler_params=_params,
    out_type=[
        jax.ShapeDtypeStruct((N_RG, N_SEG, N_FEAT), jnp.float32),
        jax.ShapeDtypeStruct((N_RG, N_SEG, N_FEAT), jnp.float32),
        jax.ShapeDtypeStruct((N_RG, N_SEG, LANES), jnp.float32),
    ],
    scratch_types=[
        pltpu.VMEM((2 * CHUNK, LANES), jnp.float32),
        pltpu.VMEM((2 * CHUNK + LANES,), jnp.int32),
        pltpu.VMEM((N_SEG, LANES), jnp.float32),
        pltpu.VMEM((N_SEG, LANES), jnp.float32),
        pltpu.VMEM((N_SEG, LANES), jnp.float32),
        pltpu.VMEM((2, LANES), jnp.float32),
        pltpu.SemaphoreType.DMA,
        pltpu.SemaphoreType.DMA,
        pltpu.SemaphoreType.DMA,
        pltpu.SemaphoreType.DMA,
    ],
)
def _phase_a(node_hbm, idx_hbm, sums_hbm, maxs_hbm, cnts_hbm,
             rowbuf, idxv, sumtab, maxtab, cnttab, accb,
             semr0, semr1, semi0, semi1):
    c = lax.axis_index("c")
    s = lax.axis_index("s")
    wid = s * 2 + c
    fg = wid % N_FG
    rg = wid // N_FG
    row0 = rg * ROWS_PER_RG
    col0 = fg * LANES

    zeros = jnp.zeros((LANES,), jnp.float32)
    ninf = jnp.full((LANES,), NEG_INF)

    def init_body(i, _):
        sumtab[i] = zeros
        maxtab[i] = ninf
        cnttab[i] = zeros
        return 0

    lax.fori_loop(0, N_SEG, init_body, 0)

    def row_body(i, carry):
        cur, cnt = carry
        sid = idxv[pl.ds(i, LANES)][0]
        changed = sid != cur
        v = rowbuf[i]
        acc_s = jnp.where(changed, v, accb[0] + v)
        acc_m = jnp.where(changed, v, jnp.maximum(accb[1], v))
        cnt = jnp.where(changed, 1.0, cnt + 1.0)
        accb[0] = acc_s
        accb[1] = acc_m
        sumtab[sid] = acc_s
        maxtab[sid] = acc_m
        cnttab[sid] = jnp.full((LANES,), cnt)
        return sid, cnt

    BLK = 16

    _sems = ((semr0, semi0), (semr1, semi1))
    _chunks = [(i * CHUNK, CHUNK) for i in range(N_FULL)]
    if TAIL_ROWS:
        _chunks.append((N_FULL * CHUNK, TAIL_ROWS))

    def _row_cp(i):
        off, size = _chunks[i]
        half = i % 2
        return pltpu.make_async_copy(
            node_hbm.at[pl.ds(row0 + off, size), pl.ds(col0, LANES)],
            rowbuf.at[pl.ds(half * CHUNK, size)], _sems[half][0])

    def _idx_cp(i):
        off, size = _chunks[i]
        half = i % 2
        return pltpu.make_async_copy(
            idx_hbm.at[pl.ds(row0 + off, size)],
            idxv.at[pl.ds(half * CHUNK, size)], _sems[half][1])

    def blk_body_for(off):
      def blk_body(b, carry):
        ids_n = idxv[pl.ds(off + (b + 1) * BLK, LANES)]
        nfirst = ids_n[0]
        nlast = ids_n[LANES - 1]
        bbase = off + b * BLK
        cur, cnt, first, last = carry

        def fast(carry):
            cur, cnt = carry
            changed = first != cur
            vs = [rowbuf[bbase + j] for j in range(BLK)]
            ss = vs
            mm = vs
            while len(ss) > 1:
                ss = [ss[2 * j] + ss[2 * j + 1] for j in range(len(ss) // 2)]
                mm = [jnp.maximum(mm[2 * j], mm[2 * j + 1])
                      for j in range(len(mm) // 2)]
            bsum, bmax = ss[0], mm[0]
            acc_s = jnp.where(changed, bsum, accb[0] + bsum)
            acc_m = jnp.where(changed, bmax, jnp.maximum(accb[1], bmax))
            cnt = jnp.where(changed, float(BLK), cnt + float(BLK))
            accb[0] = acc_s
            accb[1] = acc_m
            sumtab[first] = acc_s
            maxtab[first] = acc_m
            cnttab[first] = jnp.full((LANES,), cnt)
            return first, cnt

        def slow(carry):
            return lax.fori_loop(bbase, bbase + BLK, row_body, carry)

        cur, cnt = lax.cond(first == last, fast, slow, (cur, cnt))
        return cur, cnt, nfirst, nlast
      return blk_body

    accb[0] = zeros
    accb[1] = ninf
    _row_cp(0).start()
    _idx_cp(0).start()
    cur = jnp.int32(0)
    cnt = jnp.float32(0.0)
    for i in range(len(_chunks)):
        _, size = _chunks[i]
        boff = (i % 2) * CHUNK
        _row_cp(i).wait()
        _idx_cp(i).wait()
        if i + 1 < len(_chunks):
            _row_cp(i + 1).start()
            _idx_cp(i + 1).start()
        nb = size // BLK
        if nb:
            ids0 = idxv[pl.ds(boff, LANES)]
            carry4 = (cur, cnt, ids0[0], ids0[LANES - 1])
            cur, cnt, _, _ = lax.fori_loop(0, nb, blk_body_for(boff), carry4)
        if size % BLK:
            cur, cnt = lax.fori_loop(boff + nb * BLK, boff + size,
                                     row_body, (cur, cnt))

    pltpu.sync_copy(sumtab, sums_hbm.at[rg, :, pl.ds(col0, LANES)])
    pltpu.sync_copy(maxtab, maxs_hbm.at[rg, :, pl.ds(col0, LANES)])

    @pl.when(fg == 0)
    def _store_counts():
        pltpu.sync_copy(cnttab, cnts_hbm.at[rg])


N_WORKERS = 32
SEG_PER_W = N_SEG // N_WORKERS


@functools.partial(
    pl.kernel,
    mesh=_mesh,
    compiler_params=_params,
    out_type=jax.ShapeDtypeStruct((N_SEG * 2 * N_FG, LANES), jnp.float32),
    scratch_types=[
        pltpu.VMEM((N_RG * SEG_PER_W * N_FG, LANES), jnp.float32),
        pltpu.VMEM((N_RG * SEG_PER_W * N_FG, LANES), jnp.float32),
        pltpu.VMEM((N_RG * SEG_PER_W, LANES), jnp.float32),
        pltpu.VMEM((SEG_PER_W * 2 * N_FG, LANES), jnp.float32),
    ],
)
def _phase_b(sums_hbm, maxs_hbm, cnts_hbm, out_hbm, sbuf, mbuf, cbuf, obuf):
    c = lax.axis_index("c")
    s = lax.axis_index("s")
    wid = s * 2 + c
    seg0 = wid * SEG_PER_W
    W = SEG_PER_W * N_FG

    for rg in range(N_RG):
        pltpu.sync_copy(
            sums_hbm.at[pl.ds((rg * N_SEG + seg0) * N_FG, W)],
            sbuf.at[pl.ds(rg * W, W)])
        pltpu.sync_copy(
            maxs_hbm.at[pl.ds((rg * N_SEG + seg0) * N_FG, W)],
            mbuf.at[pl.ds(rg * W, W)])
        pltpu.sync_copy(
            cnts_hbm.at[pl.ds(rg * N_SEG + seg0, SEG_PER_W)],
            cbuf.at[pl.ds(rg * SEG_PER_W, SEG_PER_W)])

    def seg_body(k, _):
        cnt = (cbuf[k] + cbuf[SEG_PER_W + k]
               + cbuf[2 * SEG_PER_W + k] + cbuf[3 * SEG_PER_W + k])
        denom = jnp.maximum(cnt, 1.0)
        for f in range(N_FG):
            r = k * N_FG + f
            ssum = sbuf[r] + sbuf[W + r] + sbuf[2 * W + r] + sbuf[3 * W + r]
            obuf[k * 2 * N_FG + f] = ssum / denom
            mx = jnp.maximum(jnp.maximum(mbuf[r], mbuf[W + r]),
                             jnp.maximum(mbuf[2 * W + r], mbuf[3 * W + r]))
            obuf[k * 2 * N_FG + N_FG + f] = mx
        return 0

    lax.fori_loop(0, SEG_PER_W, seg_body, 0)
    pltpu.sync_copy(obuf, out_hbm.at[pl.ds(seg0 * 2 * N_FG,
                                           SEG_PER_W * 2 * N_FG)])


def kernel(node_repr, batch_idx):
    batch_idx = batch_idx.astype(jnp.int32)
    sums, maxs, cnts = _phase_a(node_repr, batch_idx)
    out = _phase_b(sums.reshape(N_RG * N_SEG * N_FG, LANES),
                   maxs.reshape(N_RG * N_SEG * N_FG, LANES),
                   cnts.reshape(N_RG * N_SEG, LANES))
    return out.reshape(N_SEG, 2 * N_FEAT)


# speedup vs baseline: 5.7922x; 1.0442x over previous
import functools

import jax
import jax.numpy as jnp
from jax import lax
from jax.experimental import pallas as pl
from jax.experimental.pallas import tpu as pltpu
from jax.experimental.pallas import tpu_sc as plsc

N_ROWS = 100000
N_FEAT = 128
N_SEG = 512
LANES = 16

N_FG = N_FEAT // LANES
N_RG = 4
ROWS_PER_RG = N_ROWS // N_RG
CHUNK = 2496
N_FULL = ROWS_PER_RG // CHUNK
TAIL_ROWS = ROWS_PER_RG - N_FULL * CHUNK

_mesh = plsc.VectorSubcoreMesh(core_axis_name="c", subcore_axis_name="s")
_params = pltpu.CompilerParams(use_tc_tiling_on_sc=False)

NEG_INF = float("-inf")


@functools.partial(
    pl.kernel,
    mesh=_mesh,
    compiler_params=_params,
    out_type=[
        jax.ShapeDtypeStruct((N_RG, N_SEG, N_FEAT), jnp.float32),
        jax.ShapeDtypeStruct((N_RG, N_SEG, N_FEAT), jnp.float32),
        jax.ShapeDtypeStruct((N_RG, N_SEG, LANES), jnp.float32),
    ],
    scratch_types=[
        pltpu.VMEM((2 * CHUNK, LANES), jnp.float32),
        pltpu.VMEM((2 * CHUNK + LANES,), jnp.int32),
        pltpu.VMEM((N_SEG, LANES), jnp.float32),
        pltpu.VMEM((N_SEG, LANES), jnp.float32),
        pltpu.VMEM((N_SEG, LANES), jnp.float32),
        pltpu.VMEM((2, LANES), jnp.float32),
        pltpu.SemaphoreType.DMA,
        pltpu.SemaphoreType.DMA,
        pltpu.SemaphoreType.DMA,
        pltpu.SemaphoreType.DMA,
    ],
)
def _phase_a(node_hbm, idx_hbm, sums_hbm, maxs_hbm, cnts_hbm,
             rowbuf, idxv, sumtab, maxtab, cnttab, accb,
             semr0, semr1, semi0, semi1):
    c = lax.axis_index("c")
    s = lax.axis_index("s")
    wid = s * 2 + c
    fg = wid % N_FG
    rg = wid // N_FG
    row0 = rg * ROWS_PER_RG
    col0 = fg * LANES

    zeros = jnp.zeros((LANES,), jnp.float32)
    ninf = jnp.full((LANES,), NEG_INF)

    def init_body(i, _):
        sumtab[i] = zeros
        maxtab[i] = ninf
        cnttab[i] = zeros
        return 0

    lax.fori_loop(0, N_SEG, init_body, 0)

    def row_body(i, carry):
        cur, cnt, sid = carry
        nsid = idxv[pl.ds(i + 1, LANES)][0]
        changed = sid != cur
        v = rowbuf[i]
        acc_s = jnp.where(changed, v, accb[0] + v)
        acc_m = jnp.where(changed, v, jnp.maximum(accb[1], v))
        cnt = jnp.where(changed, 1.0, cnt + 1.0)
        accb[0] = acc_s
        accb[1] = acc_m
        sumtab[sid] = acc_s
        maxtab[sid] = acc_m
        cnttab[sid] = jnp.full((LANES,), cnt)
        return sid, cnt, nsid

    BLK = 16

    _sems = ((semr0, semi0), (semr1, semi1))
    _chunks = [(i * CHUNK, CHUNK) for i in range(N_FULL)]
    if TAIL_ROWS:
        _chunks.append((N_FULL * CHUNK, TAIL_ROWS))

    def _row_cp(i):
        off, size = _chunks[i]
        half = i % 2
        return pltpu.make_async_copy(
            node_hbm.at[pl.ds(row0 + off, size), pl.ds(col0, LANES)],
            rowbuf.at[pl.ds(half * CHUNK, size)], _sems[half][0])

    def _idx_cp(i):
        off, size = _chunks[i]
        half = i % 2
        return pltpu.make_async_copy(
            idx_hbm.at[pl.ds(row0 + off, size)],
            idxv.at[pl.ds(half * CHUNK, size)], _sems[half][1])

    def blk_body_for(off):
      def blk_body(b, carry):
        ids_n = idxv[pl.ds(off + (b + 1) * BLK, LANES)]
        nfirst = ids_n[0]
        nlast = ids_n[LANES - 1]
        bbase = off + b * BLK
        cur, cnt, first, last = carry

        def fast(carry):
            cur, cnt = carry
            changed = first != cur
            vs = [rowbuf[bbase + j] for j in range(BLK)]
            ss = vs
            mm = vs
            while len(ss) > 1:
                ss = [ss[2 * j] + ss[2 * j + 1] for j in range(len(ss) // 2)]
                mm = [jnp.maximum(mm[2 * j], mm[2 * j + 1])
                      for j in range(len(mm) // 2)]
            bsum, bmax = ss[0], mm[0]
            acc_s = jnp.where(changed, bsum, accb[0] + bsum)
            acc_m = jnp.where(changed, bmax, jnp.maximum(accb[1], bmax))
            cnt = jnp.where(changed, float(BLK), cnt + float(BLK))
            accb[0] = acc_s
            accb[1] = acc_m
            sumtab[first] = acc_s
            maxtab[first] = acc_m
            cnttab[first] = jnp.full((LANES,), cnt)
            return first, cnt

        def slow(carry):
            cur, cnt = carry
            cur, cnt, _ = lax.fori_loop(bbase, bbase + BLK, row_body,
                                        (cur, cnt, first))
            return cur, cnt

        cur, cnt = lax.cond(first == last, fast, slow, (cur, cnt))
        return cur, cnt, nfirst, nlast
      return blk_body

    accb[0] = zeros
    accb[1] = ninf
    _row_cp(0).start()
    _idx_cp(0).start()
    cur = jnp.int32(0)
    cnt = jnp.float32(0.0)
    for i in range(len(_chunks)):
        _, size = _chunks[i]
        boff = (i % 2) * CHUNK
        _row_cp(i).wait()
        _idx_cp(i).wait()
        if i + 1 < len(_chunks):
            _row_cp(i + 1).start()
            _idx_cp(i + 1).start()
        nb = size // BLK
        ids0 = idxv[pl.ds(boff, LANES)]
        nfirst = ids0[0]
        nlast = ids0[LANES - 1]
        if nb:
            carry4 = (cur, cnt, nfirst, nlast)
            cur, cnt, nfirst, nlast = lax.fori_loop(
                0, nb, blk_body_for(boff), carry4)
        if size % BLK:
            cur, cnt, _ = lax.fori_loop(boff + nb * BLK, boff + size,
                                        row_body, (cur, cnt, nfirst))

    pltpu.sync_copy(sumtab, sums_hbm.at[rg, :, pl.ds(col0, LANES)])
    pltpu.sync_copy(maxtab, maxs_hbm.at[rg, :, pl.ds(col0, LANES)])

    @pl.when(fg == 0)
    def _store_counts():
        pltpu.sync_copy(cnttab, cnts_hbm.at[rg])


N_WORKERS = 32
SEG_PER_W = N_SEG // N_WORKERS


@functools.partial(
    pl.kernel,
    mesh=_mesh,
    compiler_params=_params,
    out_type=jax.ShapeDtypeStruct((N_SEG * 2 * N_FG, LANES), jnp.float32),
    scratch_types=[
        pltpu.VMEM((N_RG * SEG_PER_W * N_FG, LANES), jnp.float32),
        pltpu.VMEM((N_RG * SEG_PER_W * N_FG, LANES), jnp.float32),
        pltpu.VMEM((N_RG * SEG_PER_W, LANES), jnp.float32),
        pltpu.VMEM((SEG_PER_W * 2 * N_FG, LANES), jnp.float32),
    ],
)
def _phase_b(sums_hbm, maxs_hbm, cnts_hbm, out_hbm, sbuf, mbuf, cbuf, obuf):
    c = lax.axis_index("c")
    s = lax.axis_index("s")
    wid = s * 2 + c
    seg0 = wid * SEG_PER_W
    W = SEG_PER_W * N_FG

    for rg in range(N_RG):
        pltpu.sync_copy(
            sums_hbm.at[pl.ds((rg * N_SEG + seg0) * N_FG, W)],
            sbuf.at[pl.ds(rg * W, W)])
        pltpu.sync_copy(
            maxs_hbm.at[pl.ds((rg * N_SEG + seg0) * N_FG, W)],
            mbuf.at[pl.ds(rg * W, W)])
        pltpu.sync_copy(
            cnts_hbm.at[pl.ds(rg * N_SEG + seg0, SEG_PER_W)],
            cbuf.at[pl.ds(rg * SEG_PER_W, SEG_PER_W)])

    def seg_body(k, _):
        cnt = (cbuf[k] + cbuf[SEG_PER_W + k]
               + cbuf[2 * SEG_PER_W + k] + cbuf[3 * SEG_PER_W + k])
        denom = jnp.maximum(cnt, 1.0)
        for f in range(N_FG):
            r = k * N_FG + f
            ssum = sbuf[r] + sbuf[W + r] + sbuf[2 * W + r] + sbuf[3 * W + r]
            obuf[k * 2 * N_FG + f] = ssum / denom
            mx = jnp.maximum(jnp.maximum(mbuf[r], mbuf[W + r]),
                             jnp.maximum(mbuf[2 * W + r], mbuf[3 * W + r]))
            obuf[k * 2 * N_FG + N_FG + f] = mx
        return 0

    lax.fori_loop(0, SEG_PER_W, seg_body, 0)
    pltpu.sync_copy(obuf, out_hbm.at[pl.ds(seg0 * 2 * N_FG,
                                           SEG_PER_W * 2 * N_FG)])


def kernel(node_repr, batch_idx):
    batch_idx = batch_idx.astype(jnp.int32)
    sums, maxs, cnts = _phase_a(node_repr, batch_idx)
    out = _phase_b(sums.reshape(N_RG * N_SEG * N_FG, LANES),
                   maxs.reshape(N_RG * N_SEG * N_FG, LANES),
                   cnts.reshape(N_RG * N_SEG, LANES))
    return out.reshape(N_SEG, 2 * N_FEAT)
